# baseline (speedup 1.0000x reference)
import jax
import jax.numpy as jnp
from jax.experimental import pallas as pl
from jax.experimental.pallas import tpu as pltpu

_B, _S, _V = 1024, 50, 1000
_BBB = 8
_LW = 128
_TILES = [(t * _LW, min(_LW, _V - t * _LW)) for t in range((_V + _LW - 1) // _LW)]


def _tc_body(reward_ref, baseline_ref, lp_ref, seqs_ref, out_ref,
             grand_ref, cnt_ref):
    i = pl.program_id(0)

    @pl.when(i == 0)
    def _init():
        grand_ref[...] = jnp.zeros_like(grand_ref)
        cnt_ref[...] = jnp.zeros_like(cnt_ref)

    for bb in range(_BBB):
        advb = reward_ref[bb, 0] - baseline_ref[bb, 0]
        tgt = seqs_ref[bb]
        pos = tgt > 0
        w = jnp.where(pos, advb, 0.0)
        cnt_ref[...] += pos.astype(jnp.float32)
        for toff, wdt in _TILES:
            iota_t = jax.lax.broadcasted_iota(jnp.int32, (_S, wdt), 1) + toff
            tgt_t = seqs_ref[bb, :, 0:wdt]
            w_t = w[:, 0:wdt]
            eq = tgt_t == iota_t
            lp_t = lp_ref[bb, :, toff:toff + wdt]
            grand_ref[:, toff:toff + wdt] += jnp.where(eq, lp_t * w_t, 0.0)

    @pl.when(i == pl.num_programs(0) - 1)
    def _fin():
        loss_sum = -jnp.sum(grand_ref[...])
        cnt = jnp.sum(cnt_ref[...]) * (1.0 / _LW)
        out_ref[0, 0] = jnp.where(cnt > 0, loss_sum / cnt, loss_sum)


def kernel(reward, baseline, log_probs, seq):
    seq_rep = jnp.broadcast_to(seq[:, :, None], (_B, _S, _LW))
    grid = (_B // _BBB,)
    out = pl.pallas_call(
        _tc_body,
        grid=grid,
        in_specs=[
            pl.BlockSpec((_BBB, 1), lambda i: (i, 0), memory_space=pltpu.SMEM),
            pl.BlockSpec((_BBB, 1), lambda i: (i, 0), memory_space=pltpu.SMEM),
            pl.BlockSpec((_BBB, _S, _V), lambda i: (i, 0, 0)),
            pl.BlockSpec((_BBB, _S, _LW), lambda i: (i, 0, 0)),
        ],
        out_specs=pl.BlockSpec(memory_space=pltpu.SMEM),
        out_shape=jax.ShapeDtypeStruct((1, 1), jnp.float32),
        scratch_shapes=[
            pltpu.VMEM((_S, _V), jnp.float32),
            pltpu.VMEM((_S, _LW), jnp.float32),
        ],
        compiler_params=pltpu.CompilerParams(
            dimension_semantics=("arbitrary",),
        ),
    )(reward, baseline, log_probs, seq_rep)
    return out[0, 0]


# speedup vs baseline: 1.1507x; 1.1507x over previous
import jax
import jax.numpy as jnp
from jax.experimental import pallas as pl
from jax.experimental.pallas import tpu as pltpu

_B, _S, _V = 1024, 50, 1000
_BBB = 64
_LW = 128
_TILES = [(t * _LW, min(_LW, _V - t * _LW)) for t in range((_V + _LW - 1) // _LW)]


def _tc_body(reward_ref, baseline_ref, lp_ref, seqs_ref, out_ref,
             grand_ref, cnt_ref):
    i = pl.program_id(0)

    @pl.when(i == 0)
    def _init():
        grand_ref[...] = jnp.zeros_like(grand_ref)
        cnt_ref[...] = jnp.zeros_like(cnt_ref)

    for bb in range(_BBB):
        advb = reward_ref[bb, 0] - baseline_ref[bb, 0]
        tgt = seqs_ref[bb]
        pos = tgt > 0
        w = jnp.where(pos, advb, 0.0)
        cnt_ref[...] += pos.astype(jnp.float32)
        for toff, wdt in _TILES:
            iota_t = jax.lax.broadcasted_iota(jnp.int32, (_S, wdt), 1) + toff
            tgt_t = seqs_ref[bb, :, 0:wdt]
            w_t = w[:, 0:wdt]
            eq = tgt_t == iota_t
            lp_t = lp_ref[bb, :, toff:toff + wdt]
            grand_ref[:, toff:toff + wdt] += jnp.where(eq, lp_t * w_t, 0.0)

    @pl.when(i == pl.num_programs(0) - 1)
    def _fin():
        loss_sum = -jnp.sum(grand_ref[...])
        cnt = jnp.sum(cnt_ref[...]) * (1.0 / _LW)
        out_ref[0, 0] = jnp.where(cnt > 0, loss_sum / cnt, loss_sum)


def kernel(reward, baseline, log_probs, seq):
    seq_rep = jnp.broadcast_to(seq[:, :, None], (_B, _S, _LW))
    grid = (_B // _BBB,)
    out = pl.pallas_call(
        _tc_body,
        grid=grid,
        in_specs=[
            pl.BlockSpec((_BBB, 1), lambda i: (i, 0), memory_space=pltpu.SMEM),
            pl.BlockSpec((_BBB, 1), lambda i: (i, 0), memory_space=pltpu.SMEM),
            pl.BlockSpec((_BBB, _S, _V), lambda i: (i, 0, 0)),
            pl.BlockSpec((_BBB, _S, _LW), lambda i: (i, 0, 0)),
        ],
        out_specs=pl.BlockSpec(memory_space=pltpu.SMEM),
        out_shape=jax.ShapeDtypeStruct((1, 1), jnp.float32),
        scratch_shapes=[
            pltpu.VMEM((_S, _V), jnp.float32),
            pltpu.VMEM((_S, _LW), jnp.float32),
        ],
        compiler_params=pltpu.CompilerParams(
            dimension_semantics=("arbitrary",),
        ),
    )(reward, baseline, log_probs, seq_rep)
    return out[0, 0]
